# baseline (speedup 1.0000x reference)
import functools

import jax
import jax.numpy as jnp
from jax import lax
from jax.experimental import pallas as pl
from jax.experimental.pallas import tpu as pltpu
from jax.experimental.pallas import tpu_sc as plsc

NC = 2
NS = 16
NW = NC * NS
LANES = 16
W = 128
BC = 128
WIN = 8
KIDX = 2
NB = 2
BLK = 16


def _make_kernel(B, S, D, V):
    n_win = S // WIN
    blocks = S // BLK

    mesh = plsc.VectorSubcoreMesh(core_axis_name="c", subcore_axis_name="s")

    @functools.partial(
        pl.kernel,
        out_type=jax.ShapeDtypeStruct((S, D, B), jnp.float32),
        mesh=mesh,
        compiler_params=pltpu.CompilerParams(needs_layout_passes=False),
        scratch_types=[
            pltpu.VMEM((S, D), jnp.float32),
            pltpu.VMEM((KIDX, WIN, BC), jnp.int32),
            pltpu.VMEM((NB, BC, W), jnp.float32),
            pltpu.VMEM((NB, D, BC), jnp.float32),
            pltpu.SemaphoreType.DMA((KIDX,)),
            pltpu.SemaphoreType.DMA((NB,)),
            pltpu.SemaphoreType.DMA((NB,)),
        ],
    )
    def emb_kernel(idx_hbm, pos_hbm, table_hbm, out_hbm,
                   pos_v, idx_v, rows_v, trans_v, si, sg, so):
        cid = lax.axis_index("c")
        sid = lax.axis_index("s")
        wid = sid * NC + cid
        b0 = wid * BC

        pltpu.sync_copy(pos_hbm, pos_v)

        def idx_copy(w, k):
            return pltpu.make_async_copy(
                idx_hbm.at[pl.ds(w * WIN, WIN), pl.ds(b0, BC)],
                idx_v.at[k], si.at[k])

        def gather_copy(l, crel):
            return pltpu.make_async_copy(
                table_hbm.at[idx_v.at[(crel // WIN) % KIDX, crel % WIN]],
                rows_v.at[crel % NB], sg.at[crel % NB])

        def out_copy(l, crel):
            return pltpu.make_async_copy(
                trans_v.at[crel % NB],
                out_hbm.at[l, :, pl.ds(b0, BC)],
                so.at[crel % NB])

        iota = lax.iota(jnp.int32, LANES)
        dvecs = [d0 + iota for d0 in range(0, D, LANES)]

        def transpose_add(l, crel):
            bb = crel % NB
            pregs = [pos_v[l, pl.ds(d0, LANES)] for d0 in range(0, D, LANES)]

            @plsc.parallel_loop(0, BC, step=2, unroll=2)
            def _(r):
                for rr in (0, 1):
                    col = jnp.full((LANES,), 0, jnp.int32) + (r + rr)
                    for i, d0 in enumerate(range(0, D, LANES)):
                        val = rows_v[bb, r + rr, pl.ds(d0, LANES)] + pregs[i]
                        plsc.store_scatter(trans_v.at[bb],
                                           [dvecs[i], col], val)

        def chunk(blk, crel, tail):
            l = blk * BLK + crel
            bb = crel % NB
            gather_copy(l, crel).wait()

            if tail:
                out_copy(l - NB, (crel - NB) % BLK).wait()
            else:
                @pl.when(l >= NB)
                def _():
                    out_copy(l - NB, (crel - NB) % BLK).wait()

            transpose_add(l, crel)
            out_copy(l, crel).start()

            if tail:
                if l + NB < S:
                    gather_copy(l + NB, (crel + NB) % BLK).start()
            else:
                @pl.when(l + NB < S)
                def _():
                    if (crel + NB) % WIN == 0:
                        idx_copy((blk * BLK + crel + NB) // WIN,
                                 ((crel + NB) // WIN) % KIDX).wait()
                    gather_copy(l + NB, (crel + NB) % BLK).start()

            if not tail and crel % WIN == WIN - 1:
                w = blk * (BLK // WIN) + crel // WIN

                @pl.when(w + KIDX < n_win)
                def _():
                    idx_copy(w + KIDX, crel // WIN).start()

        for k in range(KIDX):
            idx_copy(k, k).start()
        idx_copy(0, 0).wait()
        for crel in range(NB):
            gather_copy(crel, crel).start()

        def block_body(blk, carry):
            for crel in range(BLK):
                chunk(blk, crel, tail=False)
            return carry
        lax.fori_loop(0, blocks, block_body, 0, unroll=False)

        for crel in range(S - blocks * BLK):
            chunk(blocks, crel, tail=True)

        for crel in range(NB):
            out_copy(S - NB + crel, (S - blocks * BLK - NB + crel)).wait()

    return emb_kernel


def kernel(x, embedding, pos_encoding):
    B, S = x.shape
    V, D = embedding.shape
    xT = jnp.swapaxes(x, 0, 1).astype(jnp.int32)
    table_p = jnp.pad(embedding, ((0, 0), (0, W - D)))
    out = _make_kernel(B, S, D, V)(xT, pos_encoding, table_p)
    return jnp.transpose(out, (2, 0, 1))
